# baseline (speedup 1.0000x reference)
import functools

import jax
import jax.numpy as jnp
from jax import lax
from jax.experimental import pallas as pl
from jax.experimental.pallas import tpu as pltpu
from jax.experimental.pallas import tpu_sc as plsc

NTAGS = 16
NTOK = 16384
NC = 2
NS = 16
NW = NC * NS
BPW = NTOK // NW
NELEM = BPW * NTAGS
CHUNK = 128
NSTR = NELEM // CHUNK
RPS = CHUNK // NTAGS
FIRE = 16

_mesh = plsc.VectorSubcoreMesh(core_axis_name="c", subcore_axis_name="s")


@functools.partial(
    pl.kernel,
    out_type=(jax.ShapeDtypeStruct((NC, NTAGS), jnp.float32),
              jax.ShapeDtypeStruct((NC, NS, NTAGS), jnp.float32)),
    mesh=_mesh,
    scratch_types=[
        pltpu.VMEM((NSTR, CHUNK), jnp.int32),
        pltpu.VMEM((NSTR, CHUNK), jnp.float32),
        pltpu.VMEM((NTAGS,), jnp.float32),
        pltpu.VMEM((NS, NTAGS), jnp.float32),
        pltpu.VMEM((NTAGS,), jnp.float32),
        pltpu.SemaphoreType.DMA,
    ],
)
def _bow_sc(elem_hbm, tflat_hbm, bias_hbm, out_hbm, scr_hbm,
            idx_v, val_v, acc_v, comb_v, bias_v, sem):
    cid = lax.axis_index("c")
    sid = lax.axis_index("s")
    wid = sid * NC + cid

    pltpu.sync_copy(elem_hbm.at[wid], idx_v)

    for r in range(NSTR // FIRE):
        copies = [
            pltpu.async_copy(tflat_hbm.at[idx_v.at[s]], val_v.at[s], sem)
            for s in range(r * FIRE, (r + 1) * FIRE)
        ]
        for c in copies:
            c.wait()

    zero16 = jnp.zeros((16,), jnp.float32)
    accs = [zero16] * 4
    for s in range(NSTR):
        for r in range(RPS):
            accs[r % 4] = accs[r % 4] + val_v[s, pl.ds(r * NTAGS, NTAGS)]
    acc_v[...] = (accs[0] + accs[1]) + (accs[2] + accs[3])

    pltpu.sync_copy(acc_v, scr_hbm.at[cid].at[sid])
    plsc.subcore_barrier()

    @pl.when(sid == 0)
    def _():
        pltpu.sync_copy(scr_hbm.at[cid], comb_v)
        pltpu.sync_copy(bias_hbm, bias_v)
        core_sum = comb_v[0, :]
        for t in range(1, NS):
            core_sum = core_sum + comb_v[t, :]

        @pl.when(cid == 0)
        def _():
            acc_v[...] = core_sum + bias_v[...]

        @pl.when(cid != 0)
        def _():
            acc_v[...] = core_sum

        pltpu.sync_copy(acc_v, out_hbm.at[cid])


def kernel(x, table, bias):
    nwords = table.shape[0]
    xi = x.reshape(NW, BPW)
    col_off = jnp.arange(NTAGS, dtype=jnp.int32) * nwords
    elem = (xi[:, :, None] + col_off[None, None, :]).reshape(NW, NSTR, CHUNK)
    tflat = table.T.reshape(-1)
    partials, _ = _bow_sc(elem, tflat, bias)
    return (partials[0] + partials[1]).reshape(1, NTAGS)


# speedup vs baseline: 1.0006x; 1.0006x over previous
import functools

import jax
import jax.numpy as jnp
from jax import lax
from jax.experimental import pallas as pl
from jax.experimental.pallas import tpu as pltpu
from jax.experimental.pallas import tpu_sc as plsc

NTAGS = 16
NTOK = 16384
NWORDS = 1000000
NC = 2
NS = 16
NW = NC * NS
BPW = NTOK // NW
NELEM = BPW * NTAGS
CHUNK = 128
NCHUNK = BPW // CHUNK
NSTR = NELEM // CHUNK
RPS = CHUNK // NTAGS
FIRE = 16

_mesh = plsc.VectorSubcoreMesh(core_axis_name="c", subcore_axis_name="s")


@functools.partial(
    pl.kernel,
    out_type=(jax.ShapeDtypeStruct((NC, NTAGS), jnp.float32),
              jax.ShapeDtypeStruct((NC, NS, NTAGS), jnp.float32)),
    mesh=_mesh,
    scratch_types=[
        pltpu.VMEM((NCHUNK, CHUNK), jnp.int32),
        pltpu.VMEM((NSTR, CHUNK), jnp.int32),
        pltpu.VMEM((NSTR, CHUNK), jnp.float32),
        pltpu.VMEM((NTAGS,), jnp.float32),
        pltpu.VMEM((NS, NTAGS), jnp.float32),
        pltpu.VMEM((NTAGS,), jnp.float32),
        pltpu.SemaphoreType.DMA,
    ],
)
def _bow_sc(x_hbm, tflat_hbm, bias_hbm, out_hbm, scr_hbm,
            x_v, idx_v, val_v, acc_v, comb_v, bias_v, sem):
    cid = lax.axis_index("c")
    sid = lax.axis_index("s")
    wid = sid * NC + cid

    pltpu.sync_copy(x_hbm.at[wid], x_v)
    col_off = lax.iota(jnp.int32, NTAGS) * NWORDS
    for j in range(NCHUNK):
        for k in range(CHUNK // NTAGS):
            xv = x_v[j, pl.ds(k * NTAGS, NTAGS)]
            for l in range(NTAGS):
                i = j * CHUNK + k * NTAGS + l
                idx_v[i // RPS, pl.ds((i % RPS) * NTAGS, NTAGS)] = col_off + xv[l]

    for r in range(NSTR // FIRE):
        copies = [
            pltpu.async_copy(tflat_hbm.at[idx_v.at[s]], val_v.at[s], sem)
            for s in range(r * FIRE, (r + 1) * FIRE)
        ]
        for c in copies:
            c.wait()

    zero16 = jnp.zeros((16,), jnp.float32)
    accs = [zero16] * 4
    for s in range(NSTR):
        for r in range(RPS):
            accs[r % 4] = accs[r % 4] + val_v[s, pl.ds(r * NTAGS, NTAGS)]
    acc_v[...] = (accs[0] + accs[1]) + (accs[2] + accs[3])

    pltpu.sync_copy(acc_v, scr_hbm.at[cid].at[sid])
    plsc.subcore_barrier()

    @pl.when(sid == 0)
    def _():
        pltpu.sync_copy(scr_hbm.at[cid], comb_v)
        pltpu.sync_copy(bias_hbm, bias_v)
        core_sum = comb_v[0, :]
        for t in range(1, NS):
            core_sum = core_sum + comb_v[t, :]

        @pl.when(cid == 0)
        def _():
            acc_v[...] = core_sum + bias_v[...]

        @pl.when(cid != 0)
        def _():
            acc_v[...] = core_sum

        pltpu.sync_copy(acc_v, out_hbm.at[cid])


def kernel(x, table, bias):
    x4 = x.reshape(NW, NCHUNK, CHUNK)
    tflat = table.T.reshape(-1)
    partials, _ = _bow_sc(x4, tflat, bias)
    return (partials[0] + partials[1]).reshape(1, NTAGS)


# speedup vs baseline: 1.0026x; 1.0021x over previous
import functools

import jax
import jax.numpy as jnp
from jax import lax
from jax.experimental import pallas as pl
from jax.experimental.pallas import tpu as pltpu
from jax.experimental.pallas import tpu_sc as plsc

NTAGS = 16
NTOK = 16384
NWORDS = 1000000
NC = 2
NS = 16
NW = NC * NS
BPW = NTOK // NW
NELEM = BPW * NTAGS
CHUNK = 128
NCHUNK = BPW // CHUNK
NGRP = CHUNK // NTAGS
NSTR = NELEM // CHUNK
RPS = CHUNK // NTAGS
FIRE = 16

_mesh = plsc.VectorSubcoreMesh(core_axis_name="c", subcore_axis_name="s")


@functools.partial(
    pl.kernel,
    out_type=(jax.ShapeDtypeStruct((NC, NTAGS), jnp.float32),
              jax.ShapeDtypeStruct((NC, NS, NTAGS), jnp.float32)),
    mesh=_mesh,
    scratch_types=[
        pltpu.VMEM((NCHUNK, CHUNK), jnp.int32),
        pltpu.VMEM((NSTR, CHUNK), jnp.int32),
        pltpu.VMEM((NSTR, CHUNK), jnp.float32),
        pltpu.VMEM((NTAGS,), jnp.float32),
        pltpu.VMEM((NS, NTAGS), jnp.float32),
        pltpu.VMEM((NTAGS,), jnp.float32),
        pltpu.SemaphoreType.DMA,
    ],
)
def _bow_sc(x_hbm, tflat_hbm, bias_hbm, out_hbm, scr_hbm,
            x_v, idx_v, val_v, acc_v, comb_v, bias_v, sem):
    cid = lax.axis_index("c")
    sid = lax.axis_index("s")
    wid = sid * NC + cid

    pltpu.sync_copy(x_hbm.at[wid], x_v)
    col_off = lax.iota(jnp.int32, NTAGS) * NWORDS

    def expand(g, carry):
        j = g // NGRP
        k = g % NGRP
        xv = x_v[j, pl.ds(k * NTAGS, NTAGS)]
        for l in range(NTAGS):
            i2 = g * NTAGS + l
            idx_v[i2 // RPS, pl.ds((i2 % RPS) * NTAGS, NTAGS)] = col_off + xv[l]
        return carry

    lax.fori_loop(0, BPW // NTAGS, expand, 0)

    def fire(r, carry):
        copies = [
            pltpu.async_copy(tflat_hbm.at[idx_v.at[r * FIRE + f]],
                             val_v.at[r * FIRE + f], sem)
            for f in range(FIRE)
        ]
        for c in copies:
            c.wait()
        return carry

    lax.fori_loop(0, NSTR // FIRE, fire, 0)

    zero16 = jnp.zeros((16,), jnp.float32)

    def accum(s, accs):
        return tuple(accs[r] + val_v[s, pl.ds(r * NTAGS, NTAGS)]
                     for r in range(RPS))

    accs = lax.fori_loop(0, NSTR, accum, (zero16,) * RPS)
    total = zero16
    for r in range(RPS):
        total = total + accs[r]
    acc_v[...] = total

    pltpu.sync_copy(acc_v, scr_hbm.at[cid].at[sid])
    plsc.subcore_barrier()

    @pl.when(sid == 0)
    def _():
        pltpu.sync_copy(scr_hbm.at[cid], comb_v)
        pltpu.sync_copy(bias_hbm, bias_v)
        core_sum = comb_v[0, :]
        for t in range(1, NS):
            core_sum = core_sum + comb_v[t, :]

        @pl.when(cid == 0)
        def _():
            acc_v[...] = core_sum + bias_v[...]

        @pl.when(cid != 0)
        def _():
            acc_v[...] = core_sum

        pltpu.sync_copy(acc_v, out_hbm.at[cid])


def kernel(x, table, bias):
    x4 = x.reshape(NW, NCHUNK, CHUNK)
    tflat = table.T.reshape(-1)
    partials, _ = _bow_sc(x4, tflat, bias)
    return (partials[0] + partials[1]).reshape(1, NTAGS)
